# Initial kernel scaffold; baseline (speedup 1.0000x reference)
import jax
import jax.numpy as jnp
from jax.experimental import pallas as pl


def kernel(view1, view2, W_enc, W_pred, center0, pre_centers, ldual0, target, epoch):
    raise NotImplementedError("write your pallas kernel here")


# speedup vs baseline: 2.7478x; 2.7478x over previous
import functools

import jax
import jax.numpy as jnp
from jax import lax
from jax.experimental import pallas as pl
from jax.experimental.pallas import tpu as pltpu
from jax.experimental.pallas import tpu_sc as plsc

B = 1024
D_IN = 2048
DIM = 128
K = 8192
T = 0.05
KT = 1024
NKT = K // KT


def _encode_body(v1, v2, we, wp, x1p_o, x2p_o, x1q_o, x2q_o, sx_o):
    x1 = jnp.dot(v1[...], we[...], preferred_element_type=jnp.float32)
    x2 = jnp.dot(v2[...], we[...], preferred_element_type=jnp.float32)
    x1q = jnp.dot(x1, wp[...], preferred_element_type=jnp.float32)
    x2q = jnp.dot(x2, wp[...], preferred_element_type=jnp.float32)

    def _norm(x):
        return x / (jnp.sqrt(jnp.sum(x * x, axis=1, keepdims=True)) + 1e-12)

    x1p = _norm(x1)
    x2p = _norm(x2)
    x1q = _norm(x1q)
    x2q = _norm(x2q)
    x1p_o[...] = x1p
    x2p_o[...] = x2p
    x1q_o[...] = x1q
    x2q_o[...] = x2q
    sx_o[...] = x1p + x2p + x1q + x2q


def _encode(view1, view2, W_enc, W_pred):
    f32 = jnp.float32
    out = pl.pallas_call(
        _encode_body,
        out_shape=[jax.ShapeDtypeStruct((B, DIM), f32)] * 5,
    )(view1, view2, W_enc, W_pred)
    return out


def _scores_body(x1p, x2p, x1q, x2q, c0, pc, ld,
                 obj_o, pret_o, lse_o, objm_o, label_o,
                 m_s, s_s, mv_s):
    i = pl.program_id(0)

    @pl.when(i == 0)
    def _():
        m_s[...] = jnp.full((8, B), -1e30, jnp.float32)
        s_s[...] = jnp.zeros((8, B), jnp.float32)
        mv_s[...] = jnp.full((1, B), 1e30, jnp.float32)

    cur = c0[...]
    cur = cur / (jnp.sqrt(jnp.sum(cur * cur, axis=0, keepdims=True)) + 1e-12)
    pre = pc[...]
    xs = (x1p[...], x2p[...], x1q[...], x2q[...])

    def _lse_update(j, logits):
        mo = m_s[j, :]
        mn = jnp.maximum(mo, jnp.max(logits, axis=1))
        s_s[j, :] = (s_s[j, :] * jnp.exp(mo - mn)
                     + jnp.sum(jnp.exp(logits - mn[:, None]), axis=1))
        m_s[j, :] = mn

    obj = jnp.zeros((B, KT), jnp.float32)
    for j in range(4):
        s = jnp.dot(xs[j], cur, preferred_element_type=jnp.float32)
        obj = obj + s
        _lse_update(j, s * (1.0 / T))
    obj = obj * -0.25
    obj_o[...] = obj

    t = obj - ld[...]
    lmin = jnp.min(t, axis=1)
    iota = lax.broadcasted_iota(jnp.int32, (B, KT), 1)
    hit = t == lmin[:, None]
    lidx = jnp.min(jnp.where(hit, iota, KT), axis=1)
    lobj = jnp.sum(jnp.where(iota == lidx[:, None], obj, 0.0), axis=1)
    upd = lmin < mv_s[0, :]
    mv_s[0, :] = jnp.where(upd, lmin, mv_s[0, :])
    label_o[0, :] = jnp.where(upd, lidx + i * KT, label_o[0, :])
    objm_o[0, :] = jnp.where(upd, lobj, objm_o[0, :])

    for j in range(4):
        s = jnp.dot(xs[j], pre, preferred_element_type=jnp.float32)
        _lse_update(4 + j, s * (1.0 / T))

    pret_o[...] = pre.T

    @pl.when(i == NKT - 1)
    def _():
        lse_o[...] = m_s[...] + jnp.log(s_s[...])


def _scores(x1p, x2p, x1q, x2q, center0, pre_centers, ldual2d):
    f32 = jnp.float32
    feat = pl.BlockSpec((B, DIM), lambda i: (0, 0))
    return pl.pallas_call(
        _scores_body,
        grid=(NKT,),
        in_specs=[feat, feat, feat, feat,
                  pl.BlockSpec((DIM, KT), lambda i: (0, i)),
                  pl.BlockSpec((DIM, KT), lambda i: (0, i)),
                  pl.BlockSpec((1, KT), lambda i: (0, i))],
        out_specs=[pl.BlockSpec((B, KT), lambda i: (0, i)),
                   pl.BlockSpec((KT, DIM), lambda i: (i, 0)),
                   pl.BlockSpec((8, B), lambda i: (0, 0)),
                   pl.BlockSpec((1, B), lambda i: (0, 0)),
                   pl.BlockSpec((1, B), lambda i: (0, 0))],
        out_shape=[jax.ShapeDtypeStruct((B, K), f32),
                   jax.ShapeDtypeStruct((K, DIM), f32),
                   jax.ShapeDtypeStruct((8, B), f32),
                   jax.ShapeDtypeStruct((1, B), f32),
                   jax.ShapeDtypeStruct((1, B), jnp.int32)],
        scratch_shapes=[pltpu.VMEM((8, B), f32),
                        pltpu.VMEM((8, B), f32),
                        pltpu.VMEM((1, B), f32)],
        compiler_params=pltpu.CompilerParams(
            dimension_semantics=("arbitrary",)),
    )(x1p, x2p, x1q, x2q, center0, pre_centers, ldual2d)


def _gather_dot(pret, label, sx):
    info = plsc.get_sparse_core_info()
    nw = info.num_cores * info.num_subcores
    bpw = B // nw
    nseg = DIM // 16
    mesh = plsc.VectorSubcoreMesh(core_axis_name="c", subcore_axis_name="s")

    @functools.partial(
        pl.kernel, mesh=mesh,
        out_type=jax.ShapeDtypeStruct((nw, 16), jnp.float32),
        scratch_types=[
            pltpu.VMEM((bpw,), jnp.int32),
            pltpu.VMEM((bpw, DIM), jnp.float32),
            pltpu.VMEM((bpw, DIM), jnp.float32),
            pltpu.VMEM((16,), jnp.float32),
            pltpu.SemaphoreType.DMA,
        ],
    )
    def k(pret_h, label_h, sx_h, out_h, idx_v, rows_v, sx_v, acc_v, sem):
        wid = lax.axis_index("s") * info.num_cores + lax.axis_index("c")
        base = wid * bpw
        pltpu.sync_copy(label_h.at[pl.ds(base, bpw)], idx_v)
        pltpu.async_copy(pret_h.at[idx_v], rows_v, sem).wait()
        pltpu.sync_copy(sx_h.at[pl.ds(base, bpw)], sx_v)
        acc = jnp.zeros((16,), jnp.float32)
        for i in range(bpw):
            for j in range(nseg):
                acc = acc + rows_v[i, pl.ds(16 * j, 16)] * sx_v[i, pl.ds(16 * j, 16)]
        acc_v[...] = acc
        pltpu.sync_copy(acc_v, out_h.at[wid])

    return k(pret, label, sx)


def kernel(view1, view2, W_enc, W_pred, center0, pre_centers, ldual0, target, epoch):
    x1p, x2p, x1q, x2q, sx = _encode(view1, view2, W_enc, W_pred)
    obj_val, pret, lse, objm, label2d = _scores(
        x1p, x2p, x1q, x2q, center0, pre_centers,
        jnp.reshape(ldual0, (1, K)))
    label = jnp.reshape(label2d, (B,))
    partials = _gather_dot(pret, label, sx)
    lse_total = jnp.sum(lse)
    vcur_sum = -4.0 * jnp.sum(objm)
    vpre_sum = jnp.sum(partials)
    loss = 0.25 * (lse_total / B - (vcur_sum + vpre_sum) / (B * T))
    return loss, label, obj_val


# speedup vs baseline: 3.0208x; 1.0993x over previous
import functools

import jax
import jax.numpy as jnp
from jax import lax
from jax.experimental import pallas as pl
from jax.experimental.pallas import tpu as pltpu
from jax.experimental.pallas import tpu_sc as plsc

B = 1024
D_IN = 2048
DIM = 128
K = 8192
T = 0.05
KT = 1024
NKT = K // KT


def _encode_body(v1, v2, we, wp, x1p_o, x2p_o, x1q_o, x2q_o, sx_o):
    x1 = jnp.dot(v1[...], we[...], preferred_element_type=jnp.float32)
    x2 = jnp.dot(v2[...], we[...], preferred_element_type=jnp.float32)
    x1q = jnp.dot(x1, wp[...], preferred_element_type=jnp.float32)
    x2q = jnp.dot(x2, wp[...], preferred_element_type=jnp.float32)

    def _norm(x):
        return x / (jnp.sqrt(jnp.sum(x * x, axis=1, keepdims=True)) + 1e-12)

    x1p = _norm(x1)
    x2p = _norm(x2)
    x1q = _norm(x1q)
    x2q = _norm(x2q)
    x1p_o[...] = x1p
    x2p_o[...] = x2p
    x1q_o[...] = x1q
    x2q_o[...] = x2q
    sx_o[...] = x1p + x2p + x1q + x2q


def _encode(view1, view2, W_enc, W_pred):
    f32 = jnp.float32
    return pl.pallas_call(
        _encode_body,
        out_shape=[jax.ShapeDtypeStruct((B, DIM), f32)] * 5,
    )(view1, view2, W_enc, W_pred)


def _scores_body(x1p, x2p, x1q, x2q, c0, pc, ld,
                 obj_o, pret_o, ldc_o, lse_o, minv_o, label_o, s_s):
    i = pl.program_id(0)

    @pl.when(i == 0)
    def _():
        s_s[...] = jnp.zeros((8, B), jnp.float32)
        minv_o[...] = jnp.full((1, B), 1e30, jnp.float32)

    cur = c0[...]
    cur = cur / (jnp.sqrt(jnp.sum(cur * cur, axis=0, keepdims=True)) + 1e-12)
    pre = pc[...]
    xs = (x1p[...], x2p[...], x1q[...], x2q[...])

    def _lse_update(j, s):
        s_s[j, :] = s_s[j, :] + jnp.sum(
            jnp.exp(s * (1.0 / T) - (1.0 / T)), axis=1)

    obj = jnp.zeros((B, KT), jnp.float32)
    for j in range(4):
        s = jnp.dot(xs[j], cur, preferred_element_type=jnp.float32)
        obj = obj + s
        _lse_update(j, s)
    obj = obj * -0.25
    obj_o[...] = obj

    t = obj - ld[...]
    lmin = jnp.min(t, axis=1)
    iota = lax.broadcasted_iota(jnp.int32, (B, KT), 1)
    lidx = jnp.min(jnp.where(t == lmin[:, None], iota, KT), axis=1)
    upd = lmin < minv_o[0, :]
    minv_o[0, :] = jnp.where(upd, lmin, minv_o[0, :])
    label_o[0, :] = jnp.where(upd, lidx + i * KT, label_o[0, :])

    for j in range(4):
        s = jnp.dot(xs[j], pre, preferred_element_type=jnp.float32)
        _lse_update(4 + j, s)

    pret_o[...] = pre.T
    lane = lax.broadcasted_iota(jnp.int32, (KT, DIM), 1)
    ldc_o[...] = jnp.where(lane == 0, jnp.transpose(ld[...]), 0.0)

    @pl.when(i == NKT - 1)
    def _():
        lse_o[...] = (1.0 / T) + jnp.log(s_s[...])


def _scores(x1p, x2p, x1q, x2q, center0, pre_centers, ldual2d):
    f32 = jnp.float32
    feat = pl.BlockSpec((B, DIM), lambda i: (0, 0))
    return pl.pallas_call(
        _scores_body,
        grid=(NKT,),
        in_specs=[feat, feat, feat, feat,
                  pl.BlockSpec((DIM, KT), lambda i: (0, i)),
                  pl.BlockSpec((DIM, KT), lambda i: (0, i)),
                  pl.BlockSpec((1, KT), lambda i: (0, i))],
        out_specs=[pl.BlockSpec((B, KT), lambda i: (0, i)),
                   pl.BlockSpec((KT, DIM), lambda i: (i, 0)),
                   pl.BlockSpec((KT, DIM), lambda i: (i, 0)),
                   pl.BlockSpec((8, B), lambda i: (0, 0)),
                   pl.BlockSpec((1, B), lambda i: (0, 0)),
                   pl.BlockSpec((1, B), lambda i: (0, 0))],
        out_shape=[jax.ShapeDtypeStruct((B, K), f32),
                   jax.ShapeDtypeStruct((K, DIM), f32),
                   jax.ShapeDtypeStruct((K, DIM), f32),
                   jax.ShapeDtypeStruct((8, B), f32),
                   jax.ShapeDtypeStruct((1, B), f32),
                   jax.ShapeDtypeStruct((1, B), jnp.int32)],
        scratch_shapes=[pltpu.VMEM((8, B), f32)],
        compiler_params=pltpu.CompilerParams(
            dimension_semantics=("arbitrary",)),
    )(x1p, x2p, x1q, x2q, center0, pre_centers, ldual2d)


def _gather_dot(pret, ldc, label, sx):
    info = plsc.get_sparse_core_info()
    nw = info.num_cores * info.num_subcores
    bpw = B // nw
    nseg = DIM // 16
    mesh = plsc.VectorSubcoreMesh(core_axis_name="c", subcore_axis_name="s")

    @functools.partial(
        pl.kernel, mesh=mesh,
        out_type=jax.ShapeDtypeStruct((nw, 16), jnp.float32),
        scratch_types=[
            pltpu.VMEM((bpw,), jnp.int32),
            pltpu.VMEM((bpw, DIM), jnp.float32),
            pltpu.VMEM((bpw, DIM), jnp.float32),
            pltpu.VMEM((bpw, DIM), jnp.float32),
            pltpu.VMEM((16,), jnp.float32),
            pltpu.SemaphoreType.DMA,
        ],
    )
    def k(pret_h, ldc_h, label_h, sx_h, out_h,
          idx_v, rows_v, ldr_v, sx_v, acc_v, sem):
        wid = lax.axis_index("s") * info.num_cores + lax.axis_index("c")
        base = wid * bpw
        pltpu.sync_copy(label_h.at[pl.ds(base, bpw)], idx_v)
        pltpu.async_copy(pret_h.at[idx_v], rows_v, sem).wait()
        pltpu.async_copy(ldc_h.at[idx_v], ldr_v, sem).wait()
        pltpu.sync_copy(sx_h.at[pl.ds(base, bpw)], sx_v)
        acc = jnp.zeros((16,), jnp.float32)
        for i in range(bpw):
            acc = acc - 4.0 * ldr_v[i, pl.ds(0, 16)]
            for j in range(nseg):
                acc = acc + rows_v[i, pl.ds(16 * j, 16)] * sx_v[i, pl.ds(16 * j, 16)]
        acc_v[...] = acc
        pltpu.sync_copy(acc_v, out_h.at[wid])

    return k(pret, ldc, label, sx)


def kernel(view1, view2, W_enc, W_pred, center0, pre_centers, ldual0, target, epoch):
    x1p, x2p, x1q, x2q, sx = _encode(view1, view2, W_enc, W_pred)
    obj_val, pret, ldc, lse, minv, label2d = _scores(
        x1p, x2p, x1q, x2q, center0, pre_centers,
        jnp.reshape(ldual0, (1, K)))
    label = jnp.reshape(label2d, (B,))
    partials = _gather_dot(pret, ldc, label, sx)
    total_v = jnp.sum(partials) - 4.0 * jnp.sum(minv)
    loss = 0.25 * (jnp.sum(lse) / B - total_v / (B * T))
    return loss, label, obj_val
